# Initial kernel scaffold; baseline (speedup 1.0000x reference)
import jax
import jax.numpy as jnp
from jax.experimental import pallas as pl


def kernel(outputs, labels):
    raise NotImplementedError("write your pallas kernel here")


# speedup vs baseline: 100.6560x; 100.6560x over previous
import functools

import jax
import jax.numpy as jnp
from jax import lax
from jax.experimental import pallas as pl
from jax.experimental.pallas import tpu as pltpu
from jax.experimental.pallas import tpu_sc as plsc

_ROWS, _COLS = 128, 32768
_TOTAL = _ROWS * _COLS
_NC, _NS, _L = 2, 16, 16
_NW = _NC * _NS
_PER_W = _TOTAL // _NW
_CHUNK = 16384
_NCHUNK = _PER_W // _CHUNK

_mesh = plsc.VectorSubcoreMesh(
    core_axis_name="c", subcore_axis_name="s", num_cores=_NC, num_subcores=_NS
)


@functools.partial(
    pl.kernel,
    out_type=jax.ShapeDtypeStruct((_NW, 3 * _L), jnp.float32),
    mesh=_mesh,
    scratch_types=[
        pltpu.VMEM((2, _CHUNK), jnp.float32),
        pltpu.VMEM((2, _CHUNK), jnp.int32),
        pltpu.VMEM((3 * _L,), jnp.float32),
        pltpu.SemaphoreType.DMA,
        pltpu.SemaphoreType.DMA,
    ],
)
def _partial_sums(x_hbm, lbl_hbm, out_hbm, xb, lb, accv, sem0, sem1):
    wid = lax.axis_index("s") * _NC + lax.axis_index("c")
    base = wid * _PER_W
    sems = (sem0, sem1)
    hx = [None, None]
    hl = [None, None]

    def start(i):
        b = i % 2
        off = base + i * _CHUNK
        hx[b] = pltpu.async_copy(x_hbm.at[pl.ds(off, _CHUNK)], xb.at[b], sems[b])
        hl[b] = pltpu.async_copy(lbl_hbm.at[pl.ds(off, _CHUNK)], lb.at[b], sems[b])

    def make_body(xv, lv):
        def body(i, accs):
            s_all, s_pos, cnt = accs
            sl = pl.ds(i * _L, _L)
            x = xv[sl]
            lf = lv[sl].astype(jnp.float32)
            sig = 1.0 / (1.0 + jnp.exp(-x))
            return (s_all + sig, s_pos + lf * sig, cnt + lf)

        return body

    start(0)
    zeros = jnp.zeros((_L,), jnp.float32)
    accs = (zeros, zeros, zeros)
    for i in range(_NCHUNK):
        b = i % 2
        hx[b].wait()
        hl[b].wait()
        if i + 1 < _NCHUNK:
            start(i + 1)
        accs = lax.fori_loop(0, _CHUNK // _L, make_body(xb.at[b], lb.at[b]), accs)

    accv[pl.ds(0, _L)] = accs[0]
    accv[pl.ds(_L, _L)] = accs[1]
    accv[pl.ds(2 * _L, _L)] = accs[2]
    pltpu.sync_copy(accv, out_hbm.at[wid])


def _finalize_body(p_ref, o_ref):
    p = p_ref[...]
    s_all = jnp.sum(p[:, 0:_L])
    s_pos = jnp.sum(p[:, _L : 2 * _L])
    k_pos = jnp.sum(p[:, 2 * _L : 3 * _L])
    k_neg = _TOTAL - k_pos
    diff = (s_all - s_pos) / k_neg - s_pos / k_pos
    o_ref[...] = jnp.maximum(diff, 0.0)[None, None]


_finalize = pl.pallas_call(
    _finalize_body,
    out_shape=jax.ShapeDtypeStruct((1, 1), jnp.float32),
)


@jax.jit
def kernel(outputs, labels):
    partials = _partial_sums(outputs.reshape(-1), labels.reshape(-1))
    return _finalize(partials)[0, 0]


# speedup vs baseline: 155.9195x; 1.5490x over previous
import functools

import jax
import jax.numpy as jnp
from jax import lax
from jax.experimental import pallas as pl
from jax.experimental.pallas import tpu as pltpu
from jax.experimental.pallas import tpu_sc as plsc

_ROWS, _COLS = 128, 32768
_TOTAL = _ROWS * _COLS
_NC, _NS, _L = 2, 16, 16
_NW = _NC * _NS
_PER_W = _TOTAL // _NW
_CHUNK = 16384
_NCHUNK = _PER_W // _CHUNK

_mesh = plsc.VectorSubcoreMesh(
    core_axis_name="c", subcore_axis_name="s", num_cores=_NC, num_subcores=_NS
)


@functools.partial(
    pl.kernel,
    out_type=jax.ShapeDtypeStruct((_NW, 3 * _L), jnp.float32),
    mesh=_mesh,
    scratch_types=[
        pltpu.VMEM((2, _CHUNK), jnp.float32),
        pltpu.VMEM((2, _CHUNK), jnp.int32),
        pltpu.VMEM((3 * _L,), jnp.float32),
        pltpu.SemaphoreType.DMA,
        pltpu.SemaphoreType.DMA,
    ],
)
def _partial_sums(x_hbm, lbl_hbm, out_hbm, xb, lb, accv, sem0, sem1):
    wid = lax.axis_index("s") * _NC + lax.axis_index("c")
    rows_per_w = _ROWS // _NW
    chunks_per_row = _COLS // _CHUNK
    row0 = wid * rows_per_w
    sems = (sem0, sem1)
    hx = [None, None]
    hl = [None, None]

    def start(i):
        b = i % 2
        row = row0 + i // chunks_per_row
        col = (i % chunks_per_row) * _CHUNK
        hx[b] = pltpu.async_copy(
            x_hbm.at[row, pl.ds(col, _CHUNK)], xb.at[b], sems[b]
        )
        hl[b] = pltpu.async_copy(
            lbl_hbm.at[row, pl.ds(col, _CHUNK)], lb.at[b], sems[b]
        )

    def make_body(xv, lv):
        def body(i, accs):
            s_all, s_pos, cnt = accs
            sl = pl.ds(i * _L, _L)
            x = xv[sl]
            lf = lv[sl].astype(jnp.float32)
            sig = 1.0 / (1.0 + jnp.exp(-x))
            return (s_all + sig, s_pos + lf * sig, cnt + lf)

        return body

    start(0)
    zeros = jnp.zeros((_L,), jnp.float32)
    accs = (zeros, zeros, zeros)
    for i in range(_NCHUNK):
        b = i % 2
        hx[b].wait()
        hl[b].wait()
        if i + 1 < _NCHUNK:
            start(i + 1)
        accs = lax.fori_loop(0, _CHUNK // _L, make_body(xb.at[b], lb.at[b]), accs)

    accv[pl.ds(0, _L)] = accs[0]
    accv[pl.ds(_L, _L)] = accs[1]
    accv[pl.ds(2 * _L, _L)] = accs[2]
    pltpu.sync_copy(accv, out_hbm.at[wid])


def _finalize_body(p_ref, o_ref):
    p = p_ref[...]
    s_all = jnp.sum(p[:, 0:_L])
    s_pos = jnp.sum(p[:, _L : 2 * _L])
    k_pos = jnp.sum(p[:, 2 * _L : 3 * _L])
    k_neg = _TOTAL - k_pos
    diff = (s_all - s_pos) / k_neg - s_pos / k_pos
    o_ref[...] = jnp.maximum(diff, 0.0)[None, None]


_finalize = pl.pallas_call(
    _finalize_body,
    out_shape=jax.ShapeDtypeStruct((1, 1), jnp.float32),
)


@jax.jit
def kernel(outputs, labels):
    partials = _partial_sums(outputs, labels)
    return _finalize(partials)[0, 0]
